# baseline (speedup 1.0000x reference)
import functools

import jax
import jax.numpy as jnp
from jax import lax
from jax.experimental import pallas as pl
from jax.experimental.pallas import tpu as pltpu
from jax.experimental.pallas import tpu_sc as plsc

_B = 16384
_A = 64
_S = 1000000
_NC = 2
_NS = 16
_NW = _NC * _NS
_BPW = _B // _NW
_L = 16
_WIN = 512
_WSH = 15
_SPAD = 1000064
_NBUF = 3
_NBLK = _BPW // _L

_LN2 = 0.6931471805599453
_SQRT2 = 1.4142135623730951


def _ln(x):
    bits = plsc.bitcast(x, jnp.int32)
    e = lax.shift_right_logical(bits, 23) - 127
    mbits = lax.bitwise_or(lax.bitwise_and(bits, 0x7FFFFF), 0x3F800000)
    m = plsc.bitcast(mbits, jnp.float32)
    big = m > _SQRT2
    e = jnp.where(big, e + 1, e)
    m = jnp.where(big, m * 0.5, m)
    t = (m - 1.0) / (m + 1.0)
    t2 = t * t
    p = t * (2.0 + t2 * (2.0 / 3.0 + t2 * (2.0 / 5.0 + t2 * (2.0 / 7.0))))
    return e.astype(jnp.float32) * _LN2 + p


def _sget(ref, i):
    return ref[pl.ds(i, _L)][0]


def _col0(w):
    return pl.multiple_of(jnp.minimum(w * _WIN, _SPAD - _WIN), 128)


def _win_dma(pt_hbm, win_ref, sem, w):
    return pltpu.async_copy(pt_hbm.at[:, pl.ds(_col0(w), _WIN)], win_ref, sem)


def _sc_body(pt_hbm, keys_hbm, out_hbm, keys_v, dw_v, rs_v, out_v, *ws):
    wid = lax.axis_index("s") * _NC + lax.axis_index("c")
    wins = list(ws[:_NBUF])
    sems = list(ws[_NBUF:])

    keys_v[pl.ds(0, _L)] = jnp.full((_L,), -1, jnp.int32)
    pltpu.sync_copy(keys_hbm.at[pl.ds(wid * _BPW, _BPW)],
                    keys_v.at[pl.ds(8, _BPW)])

    iota = lax.broadcasted_iota(jnp.int32, (_L,), 0)

    def scan_blk(i, off):
        kv = keys_v[pl.ds(8 + i * _L, _L)]
        kp = keys_v[pl.ds(7 + i * _L, _L)]
        w = lax.shift_right_logical(kv, _WSH)
        wp = lax.shift_right_arithmetic(kp, _WSH)
        bm = w != wp
        plsc.store_compressed(dw_v.at[pl.ds(off, _L)], w, mask=bm)
        plsc.store_compressed(rs_v.at[pl.ds(off, _L)], i * _L + iota, mask=bm)
        pc = plsc.all_reduce_population_count(bm)
        return off + lax.reduce_max(pc, (0,))

    nruns = lax.fori_loop(0, _NBLK, scan_blk, jnp.int32(0))
    rs_v[pl.ds(nruns, _L)] = jnp.full((_L,), _BPW, jnp.int32)

    for i in range(_NBUF - 1):
        @pl.when(i < nruns)
        def _(i=i):
            _win_dma(pt_hbm, wins[i], sems[i], _sget(dw_v, i))

    ce_acc = jnp.zeros((_L,), jnp.float32)
    cnt_acc = jnp.zeros((_L,), jnp.float32)

    ngrp = (nruns + _NBUF - 1) // _NBUF

    def grp(g, carry):
        ce_a, cnt_a = carry
        for b in range(_NBUF):
            r = g * _NBUF + b

            @pl.when(r + _NBUF - 1 < nruns)
            def _(r=r, b=b):
                _win_dma(pt_hbm, wins[(b + _NBUF - 1) % _NBUF],
                         sems[(b + _NBUF - 1) % _NBUF],
                         _sget(dw_v, r + _NBUF - 1))

            def run_body(r=r, b=b, ce_a=ce_a, cnt_a=cnt_a):
                pltpu.make_async_copy(
                    pt_hbm.at[:, pl.ds(0, _WIN)], wins[b], sems[b]).wait()
                base = _sget(rs_v, r)
                rend = _sget(rs_v, r + 1)
                c0 = _col0(_sget(dw_v, r))
                nsub = (rend - base + _L - 1) // _L

                def sub(j, c2):
                    ce_s, cnt_s = c2
                    off = base + j * _L
                    kv = keys_v[pl.ds(8 + off, _L)]
                    lanes = iota < (rend - off)
                    col = lax.shift_right_logical(kv, 6) - c0
                    albl = lax.bitwise_and(kv, _A - 1)
                    m = jnp.full((_L,), -3.0e38, jnp.float32)
                    amax = jnp.zeros((_L,), jnp.int32)
                    s = jnp.zeros((_L,), jnp.float32)
                    for a in range(_A):
                        arow = jnp.full((_L,), a, jnp.int32)
                        v = plsc.load_gather(wins[b], [arow, col], mask=lanes)
                        upd = v > m
                        amax = jnp.where(upd, a, amax)
                        m = jnp.maximum(m, v)
                        s = s + jnp.exp(v)
                    la = plsc.load_gather(wins[b], [albl, col], mask=lanes)
                    ce = _ln(s) - la
                    ok = jnp.logical_and(lanes, amax == albl)
                    ce_s = ce_s + jnp.where(lanes, ce, 0.0)
                    cnt_s = cnt_s + jnp.where(ok, 1.0, 0.0)
                    return ce_s, cnt_s

                return lax.fori_loop(0, nsub, sub, (ce_a, cnt_a))

            ce_a, cnt_a = lax.cond(
                r < nruns, run_body, lambda ce_a=ce_a, cnt_a=cnt_a:
                (ce_a, cnt_a))
        return ce_a, cnt_a

    ce_acc, cnt_acc = lax.fori_loop(0, ngrp, grp, (ce_acc, cnt_acc))

    out_v[pl.ds(0, _L)] = ce_acc
    out_v[pl.ds(_L, _L)] = cnt_acc
    pltpu.sync_copy(out_v, out_hbm.at[wid])


_sc_ce = functools.partial(
    pl.kernel,
    out_type=jax.ShapeDtypeStruct((_NW, 2 * _L), jnp.float32),
    mesh=plsc.VectorSubcoreMesh(
        core_axis_name="c", subcore_axis_name="s",
        num_cores=_NC, num_subcores=_NS),
    compiler_params=pltpu.CompilerParams(
        needs_layout_passes=False, use_tc_tiling_on_sc=True),
    scratch_types=[
        pltpu.VMEM((8 + _BPW + 8,), jnp.int32),
        pltpu.VMEM((_BPW + 24,), jnp.int32),
        pltpu.VMEM((_BPW + 24,), jnp.int32),
        pltpu.VMEM((2 * _L,), jnp.float32),
    ] + [pltpu.VMEM((_A, _WIN), jnp.float32)] * _NBUF
      + [pltpu.SemaphoreType.DMA] * _NBUF,
)(_sc_body)


def kernel(states, actions, rewards, policy):
    del rewards
    keys = lax.sort(states * _A + actions, is_stable=False)
    out = _sc_ce(policy.T, keys)
    r = out.reshape(_NW, 2, _L)
    loss = jnp.sum(r[:, 0, :]) / _B
    acc = jnp.sum(r[:, 1, :]) / _B
    return (loss, acc)


# speedup vs baseline: 1.0972x; 1.0972x over previous
import functools

import jax
import jax.numpy as jnp
from jax import lax
from jax.experimental import pallas as pl
from jax.experimental.pallas import tpu as pltpu
from jax.experimental.pallas import tpu_sc as plsc

_B = 16384
_A = 64
_S = 1000000
_NC = 2
_NS = 16
_NW = _NC * _NS
_BPW = _B // _NW
_L = 16
_WIN = 256
_WSH = 14
_SPAD = 1000064
_NBUF = 6
_NBLK = _BPW // _L

_LN2 = 0.6931471805599453
_SQRT2 = 1.4142135623730951


def _ln(x):
    bits = plsc.bitcast(x, jnp.int32)
    e = lax.shift_right_logical(bits, 23) - 127
    mbits = lax.bitwise_or(lax.bitwise_and(bits, 0x7FFFFF), 0x3F800000)
    m = plsc.bitcast(mbits, jnp.float32)
    big = m > _SQRT2
    e = jnp.where(big, e + 1, e)
    m = jnp.where(big, m * 0.5, m)
    t = (m - 1.0) / (m + 1.0)
    t2 = t * t
    p = t * (2.0 + t2 * (2.0 / 3.0 + t2 * (2.0 / 5.0 + t2 * (2.0 / 7.0))))
    return e.astype(jnp.float32) * _LN2 + p


def _sget(ref, i):
    return ref[pl.ds(i, _L)][0]


def _col0(w):
    return pl.multiple_of(jnp.minimum(w * _WIN, _SPAD - _WIN), 128)


def _win_dma(pt_hbm, win_ref, sem, w):
    return pltpu.async_copy(pt_hbm.at[:, pl.ds(_col0(w), _WIN)], win_ref, sem)


def _sc_body(pt_hbm, keys_hbm, out_hbm, keys_v, dw_v, rs_v, out_v, *ws):
    wid = lax.axis_index("s") * _NC + lax.axis_index("c")
    wins = list(ws[:_NBUF])
    sems = list(ws[_NBUF:])

    keys_v[pl.ds(0, _L)] = jnp.full((_L,), -1, jnp.int32)
    pltpu.sync_copy(keys_hbm.at[pl.ds(wid * _BPW, _BPW)],
                    keys_v.at[pl.ds(8, _BPW)])

    iota = lax.broadcasted_iota(jnp.int32, (_L,), 0)

    def scan_blk(i, off):
        kv = keys_v[pl.ds(8 + i * _L, _L)]
        kp = keys_v[pl.ds(7 + i * _L, _L)]
        w = lax.shift_right_logical(kv, _WSH)
        wp = lax.shift_right_arithmetic(kp, _WSH)
        bm = w != wp
        plsc.store_compressed(dw_v.at[pl.ds(off, _L)], w, mask=bm)
        plsc.store_compressed(rs_v.at[pl.ds(off, _L)], i * _L + iota, mask=bm)
        pc = plsc.all_reduce_population_count(bm)
        return off + lax.reduce_max(pc, (0,))

    nruns = lax.fori_loop(0, _NBLK, scan_blk, jnp.int32(0))
    rs_v[pl.ds(nruns, _L)] = jnp.full((_L,), _BPW, jnp.int32)

    for i in range(_NBUF - 1):
        @pl.when(i < nruns)
        def _(i=i):
            _win_dma(pt_hbm, wins[i], sems[i], _sget(dw_v, i))

    ce_acc = jnp.zeros((_L,), jnp.float32)
    cnt_acc = jnp.zeros((_L,), jnp.float32)

    ngrp = (nruns + _NBUF - 1) // _NBUF

    def grp(g, carry):
        ce_a, cnt_a = carry
        for b in range(_NBUF):
            r = g * _NBUF + b

            @pl.when(r + _NBUF - 1 < nruns)
            def _(r=r, b=b):
                _win_dma(pt_hbm, wins[(b + _NBUF - 1) % _NBUF],
                         sems[(b + _NBUF - 1) % _NBUF],
                         _sget(dw_v, r + _NBUF - 1))

            def run_body(r=r, b=b, ce_a=ce_a, cnt_a=cnt_a):
                pltpu.make_async_copy(
                    pt_hbm.at[:, pl.ds(0, _WIN)], wins[b], sems[b]).wait()
                base = _sget(rs_v, r)
                rend = _sget(rs_v, r + 1)
                c0 = _col0(_sget(dw_v, r))
                nsub = (rend - base + _L - 1) // _L

                def sub(j, c2):
                    ce_s, cnt_s = c2
                    off = base + j * _L
                    kv = keys_v[pl.ds(8 + off, _L)]
                    lanes = iota < (rend - off)
                    col = lax.shift_right_logical(kv, 6) - c0
                    albl = lax.bitwise_and(kv, _A - 1)
                    m = jnp.full((_L,), -3.0e38, jnp.float32)
                    amax = jnp.zeros((_L,), jnp.int32)
                    s = jnp.zeros((_L,), jnp.float32)
                    for a in range(_A):
                        arow = jnp.full((_L,), a, jnp.int32)
                        v = plsc.load_gather(wins[b], [arow, col], mask=lanes)
                        upd = v > m
                        amax = jnp.where(upd, a, amax)
                        m = jnp.maximum(m, v)
                        s = s + jnp.exp(v)
                    la = plsc.load_gather(wins[b], [albl, col], mask=lanes)
                    ce = _ln(s) - la
                    ok = jnp.logical_and(lanes, amax == albl)
                    ce_s = ce_s + jnp.where(lanes, ce, 0.0)
                    cnt_s = cnt_s + jnp.where(ok, 1.0, 0.0)
                    return ce_s, cnt_s

                return lax.fori_loop(0, nsub, sub, (ce_a, cnt_a))

            ce_a, cnt_a = lax.cond(
                r < nruns, run_body, lambda ce_a=ce_a, cnt_a=cnt_a:
                (ce_a, cnt_a))
        return ce_a, cnt_a

    ce_acc, cnt_acc = lax.fori_loop(0, ngrp, grp, (ce_acc, cnt_acc))

    out_v[pl.ds(0, _L)] = ce_acc
    out_v[pl.ds(_L, _L)] = cnt_acc
    pltpu.sync_copy(out_v, out_hbm.at[wid])


_sc_ce = functools.partial(
    pl.kernel,
    out_type=jax.ShapeDtypeStruct((_NW, 2 * _L), jnp.float32),
    mesh=plsc.VectorSubcoreMesh(
        core_axis_name="c", subcore_axis_name="s",
        num_cores=_NC, num_subcores=_NS),
    compiler_params=pltpu.CompilerParams(
        needs_layout_passes=False, use_tc_tiling_on_sc=True),
    scratch_types=[
        pltpu.VMEM((8 + _BPW + 8,), jnp.int32),
        pltpu.VMEM((_BPW + 24,), jnp.int32),
        pltpu.VMEM((_BPW + 24,), jnp.int32),
        pltpu.VMEM((2 * _L,), jnp.float32),
    ] + [pltpu.VMEM((_A, _WIN), jnp.float32)] * _NBUF
      + [pltpu.SemaphoreType.DMA] * _NBUF,
)(_sc_body)


def kernel(states, actions, rewards, policy):
    del rewards
    keys = lax.sort(states * _A + actions, is_stable=False)
    out = _sc_ce(policy.T, keys)
    r = out.reshape(_NW, 2, _L)
    loss = jnp.sum(r[:, 0, :]) / _B
    acc = jnp.sum(r[:, 1, :]) / _B
    return (loss, acc)
